# baseline (speedup 1.0000x reference)
import functools

import jax
import jax.numpy as jnp
from jax import lax
from jax.experimental import pallas as pl
from jax.experimental.pallas import tpu as pltpu
from jax.experimental.pallas import tpu_sc as plsc

NC = 2
NS = 16
NW = NC * NS

CHUNK = 128
GROUP = 13

CB = 16384
OB = CB // 8


def _detile_body(in_ref, out_ref):
    t = in_ref[...]
    out_ref[...] = t.reshape(16, OB, 8).transpose(1, 2, 0).reshape(OB, 128)


def _make_detile(v, d):
    grid = (v + CB - 1) // CB
    return pl.pallas_call(
        _detile_body,
        grid=(grid,),
        in_specs=[pl.BlockSpec((d, CB), lambda i: (0, i))],
        out_specs=pl.BlockSpec((OB, 128), lambda i: (i, 0)),
        out_shape=jax.ShapeDtypeStruct((v * d // 128, 128), jnp.float32),
    )


def _make_gather(total, v, d):
    per_w = total // NW
    n_chunk = per_w // CHUNK
    n_group = n_chunk // GROUP
    rows = GROUP * CHUNK
    assert per_w * NW == total and n_chunk * CHUNK == per_w
    assert n_group * GROUP == n_chunk

    mesh = plsc.VectorSubcoreMesh(core_axis_name="c", subcore_axis_name="s")

    @functools.partial(
        pl.kernel,
        mesh=mesh,
        compiler_params=pltpu.CompilerParams(use_tc_tiling_on_sc=False),
        out_type=jax.ShapeDtypeStruct((total, d), jnp.float32),
        scratch_types=[
            pltpu.VMEM((n_chunk, CHUNK), jnp.int32),
            pltpu.VMEM((rows, d), jnp.float32),
            pltpu.SemaphoreType.DMA,
        ],
    )
    def gather_kernel(idx_hbm, tab_hbm, out_hbm, idx_v, rows_v, gsem):
        wid = lax.axis_index("s") * NC + lax.axis_index("c")
        pltpu.sync_copy(idx_hbm.at[pl.ds(wid * n_chunk, n_chunk)], idx_v)

        def group_body(g, carry):
            handles = []
            for b in range(GROUP):
                h = pltpu.async_copy(
                    tab_hbm.at[idx_v.at[g * GROUP + b]],
                    rows_v.at[pl.ds(b * CHUNK, CHUNK)],
                    gsem,
                )
                handles.append(h)
            for h in handles:
                h.wait()
            pltpu.sync_copy(
                rows_v, out_hbm.at[pl.ds(wid * per_w + g * rows, rows)])
            return carry

        lax.fori_loop(0, n_group, group_body, 0)

    return gather_kernel


def kernel(x, table):
    b, f = x.shape
    v, d = table.shape
    total = b * f
    tab_lin = _make_detile(v, d)(table.T).reshape(v, d)
    idx = x.reshape(total // CHUNK, CHUNK).astype(jnp.int32)
    out = _make_gather(total, v, d)(idx, tab_lin)
    return out.reshape(b, f * d)
